# baseline (speedup 1.0000x reference)
import functools

import jax
import jax.numpy as jnp
from jax import lax
from jax.experimental import pallas as pl
from jax.experimental.pallas import tpu as pltpu
from jax.experimental.pallas import tpu_sc as plsc

B = 4
H = 224
W = 224
D = 256
NUM_CLASSES = 1000
NUM_STEPS = 1000

HWPROD = H * W
ROWS_PER_PLANE = HWPROD // D
HCHUNK = ROWS_PER_PLANE // 2
IDXP = 112
HH = H // 2
NLANES = 16
WND = 8

_SC_INFO = plsc.get_sparse_core_info()
NW = _SC_INFO.num_cores * _SC_INFO.num_subcores
B2 = B // 2
NPLANES = B2 * D
PPW = NPLANES // NW
CPW = 2 * PPW


def _cond2d_body(ts_ref, cl_ref, tt_ref, wp_ref, bp_ref, ct_ref, out_ref):
    t_rows = jnp.concatenate(
        [tt_ref[pl.ds(ts_ref[b], 1), :] for b in range(B)], axis=0)
    c_rows = jnp.concatenate(
        [ct_ref[pl.ds(cl_ref[b], 1), :] for b in range(B)], axis=0)
    t_emb = jnp.dot(t_rows, wp_ref[...], preferred_element_type=jnp.float32)
    out_ref[...] = t_emb + bp_ref[...] + c_rows


def _cond2d(time_step, class_label, time_table, W_proj, b_proj, class_table):
    return pl.pallas_call(
        _cond2d_body,
        out_shape=jax.ShapeDtypeStruct((B, D), jnp.float32),
        in_specs=[
            pl.BlockSpec(memory_space=pltpu.SMEM),
            pl.BlockSpec(memory_space=pltpu.SMEM),
            pl.BlockSpec(memory_space=pltpu.VMEM),
            pl.BlockSpec(memory_space=pltpu.VMEM),
            pl.BlockSpec(memory_space=pltpu.VMEM),
            pl.BlockSpec(memory_space=pltpu.VMEM),
        ],
    )(time_step, class_label, time_table, W_proj,
      b_proj.reshape(1, D), class_table)


def _sc_body(table_hbm, idx_hbm, scal_hbm, out_hbm, idx_v, scal_v,
             gb0, gb1, am0, am1, gsem0, gsem1, ssem0, ssem1):
    wid = lax.axis_index("s") * _SC_INFO.num_cores + lax.axis_index("c")
    pltpu.sync_copy(idx_hbm.at[pl.ds(wid * CPW, CPW)], idx_v)
    pltpu.sync_copy(scal_hbm.at[pl.ds(wid * PPW, PPW)], scal_v)
    gbufs = (gb0, gb1)
    ams = (am0, am1)
    gsems = (gsem0, gsem1)
    ssems = (ssem0, ssem1)

    pltpu.async_copy(table_hbm.at[idx_v.at[0]], gb0, gsem0)

    def pair_body(jj, carry):
        g = wid * PPW + jj
        b = g >> 8
        d = g & (D - 1)
        svec = scal_v[jj]
        for par in (0, 1):
            j = 2 * jj + par
            gb = gbufs[par]
            am = ams[par]
            pltpu.make_async_copy(table_hbm.at[idx_v.at[j]], gb,
                                  gsems[par]).wait()

            @pl.when(j + 1 < CPW)
            def _():
                pltpu.async_copy(table_hbm.at[idx_v.at[j + 1]],
                                 gbufs[1 - par], gsems[1 - par])

            @pl.when(jj > 0)
            def _():
                pltpu.make_async_copy(
                    out_hbm.at[0, 0, pl.ds(0, HH)], am, ssems[par]).wait()

            def block_body(bh, c2):
                r0 = 7 * bh
                h0 = 8 * bh
                vals = {}
                for k in range(112 + WND):
                    if k < 112:
                        vals[k] = gb[r0 + k // 16,
                                     pl.ds((16 * k) % D, NLANES)]
                    kk = k - WND
                    if kk >= 0:
                        am[h0 + kk // 14, pl.ds((16 * kk) % W, NLANES)] = (
                            vals.pop(kk) + svec)
                return c2

            lax.fori_loop(0, HH // 8, block_body, 0)
            pltpu.async_copy(am, out_hbm.at[b, d, pl.ds(par * HH, HH)],
                             ssems[par])
        return carry

    lax.fori_loop(0, PPW, pair_body, 0)
    pltpu.make_async_copy(out_hbm.at[0, 0, pl.ds(0, HH)], am0, ssem0).wait()
    pltpu.make_async_copy(out_hbm.at[0, 0, pl.ds(0, HH)], am1, ssem1).wait()


@functools.partial(
    pl.kernel,
    mesh=plsc.VectorSubcoreMesh(core_axis_name="c", subcore_axis_name="s"),
    out_type=jax.ShapeDtypeStruct((B2, D, H, W), jnp.float32),
    scratch_types=[
        pltpu.VMEM((CPW, IDXP), jnp.int32),
        pltpu.VMEM((PPW, NLANES), jnp.float32),
        pltpu.VMEM((IDXP, D), jnp.float32),
        pltpu.VMEM((IDXP, D), jnp.float32),
        pltpu.VMEM((HH, W), jnp.float32),
        pltpu.VMEM((HH, W), jnp.float32),
        pltpu.SemaphoreType.DMA,
        pltpu.SemaphoreType.DMA,
        pltpu.SemaphoreType.DMA,
        pltpu.SemaphoreType.DMA,
    ],
)
def _sc_gather_add(table_hbm, idx_hbm, scal_hbm, out_hbm, idx_v, scal_v,
                   gb0, gb1, am0, am1, gsem0, gsem1, ssem0, ssem1):
    _sc_body(table_hbm, idx_hbm, scal_hbm, out_hbm, idx_v, scal_v,
             gb0, gb1, am0, am1, gsem0, gsem1, ssem0, ssem1)


def kernel(time_step, class_label, cond_3d, time_table, W_proj, b_proj,
           class_table):
    cond2d = _cond2d(time_step, class_label, time_table, W_proj, b_proj,
                     class_table)
    scal = jnp.broadcast_to(
        cond2d.reshape(B * D, 1), (B * D, NLANES))
    flat2 = cond_3d.reshape(2 * B * D, HCHUNK)
    idx = jnp.concatenate(
        [flat2, flat2[:, :IDXP - HCHUNK]], axis=1)
    nc2 = B2 * D * 2
    out_a = _sc_gather_add(class_table, idx[:nc2], scal[:B2 * D])
    out_b = _sc_gather_add(class_table, idx[nc2:], scal[B2 * D:])
    return jnp.concatenate([out_a, out_b], axis=0)


# speedup vs baseline: 1.2346x; 1.2346x over previous
import functools

import jax
import jax.numpy as jnp
from jax import lax
from jax.experimental import pallas as pl
from jax.experimental.pallas import tpu as pltpu
from jax.experimental.pallas import tpu_sc as plsc

B = 4
H = 224
W = 224
D = 256
NUM_CLASSES = 1000
NUM_STEPS = 1000

HWPROD = H * W
ROWS_PER_PLANE = HWPROD // D
HCHUNK = ROWS_PER_PLANE // 2
IDXP = 112
HH = H // 2
NLANES = 16
WND = 8

_SC_INFO = plsc.get_sparse_core_info()
NW = _SC_INFO.num_cores * _SC_INFO.num_subcores
NPLANES = B * D
PPW = NPLANES // NW
CPW = 2 * PPW


def _cond2d_body(ts_ref, cl_ref, tt_ref, wp_ref, bp_ref, ct_ref, out_ref):
    t_rows = jnp.concatenate(
        [tt_ref[pl.ds(ts_ref[b], 1), :] for b in range(B)], axis=0)
    c_rows = jnp.concatenate(
        [ct_ref[pl.ds(cl_ref[b], 1), :] for b in range(B)], axis=0)
    t_emb = jnp.dot(t_rows, wp_ref[...], preferred_element_type=jnp.float32)
    out_ref[...] = t_emb + bp_ref[...] + c_rows


def _cond2d(time_step, class_label, time_table, W_proj, b_proj, class_table):
    return pl.pallas_call(
        _cond2d_body,
        out_shape=jax.ShapeDtypeStruct((B, D), jnp.float32),
        in_specs=[
            pl.BlockSpec(memory_space=pltpu.SMEM),
            pl.BlockSpec(memory_space=pltpu.SMEM),
            pl.BlockSpec(memory_space=pltpu.VMEM),
            pl.BlockSpec(memory_space=pltpu.VMEM),
            pl.BlockSpec(memory_space=pltpu.VMEM),
            pl.BlockSpec(memory_space=pltpu.VMEM),
        ],
    )(time_step, class_label, time_table, W_proj,
      b_proj.reshape(1, D), class_table)


def _sc_body(table_hbm, idx_hbm, scal_hbm, out_hbm, idx_v, scal_v,
             gb0, gb1, am0, am1, gsem0, gsem1, ssem0, ssem1):
    wid = lax.axis_index("s") * _SC_INFO.num_cores + lax.axis_index("c")
    pltpu.sync_copy(idx_hbm.at[pl.ds(wid * CPW, CPW)], idx_v)
    pltpu.sync_copy(scal_hbm.at[pl.ds(wid * PPW, PPW)], scal_v)
    gbufs = (gb0, gb1)
    ams = (am0, am1)
    gsems = (gsem0, gsem1)
    ssems = (ssem0, ssem1)

    pltpu.async_copy(table_hbm.at[idx_v.at[0]], gb0, gsem0)

    def pair_body(jj, carry):
        g = wid * PPW + jj
        b = g >> 8
        d = g & (D - 1)
        svec = scal_v[jj]
        for par in (0, 1):
            j = 2 * jj + par
            gb = gbufs[par]
            am = ams[par]
            pltpu.make_async_copy(table_hbm.at[idx_v.at[j]], gb,
                                  gsems[par]).wait()

            @pl.when(j + 1 < CPW)
            def _():
                pltpu.async_copy(table_hbm.at[idx_v.at[j + 1]],
                                 gbufs[1 - par], gsems[1 - par])

            @pl.when(jj > 0)
            def _():
                pltpu.make_async_copy(
                    out_hbm.at[0, 0, pl.ds(0, HH)], am, ssems[par]).wait()

            def block_body(bh, c2):
                r0 = 7 * bh
                h0 = 8 * bh
                vals = {}
                for k in range(112 + WND):
                    if k < 112:
                        vals[k] = gb[r0 + k // 16,
                                     pl.ds((16 * k) % D, NLANES)]
                    kk = k - WND
                    if kk >= 0:
                        am[h0 + kk // 14, pl.ds((16 * kk) % W, NLANES)] = (
                            vals.pop(kk) + svec)
                return c2

            lax.fori_loop(0, HH // 8, block_body, 0)
            pltpu.async_copy(am, out_hbm.at[b, d, pl.ds(par * HH, HH)],
                             ssems[par])
        return carry

    lax.fori_loop(0, PPW, pair_body, 0)
    pltpu.make_async_copy(out_hbm.at[0, 0, pl.ds(0, HH)], am0, ssem0).wait()
    pltpu.make_async_copy(out_hbm.at[0, 0, pl.ds(0, HH)], am1, ssem1).wait()


@functools.partial(
    pl.kernel,
    mesh=plsc.VectorSubcoreMesh(core_axis_name="c", subcore_axis_name="s"),
    out_type=jax.ShapeDtypeStruct((B, D, H, W), jnp.float32),
    scratch_types=[
        pltpu.VMEM((CPW, IDXP), jnp.int32),
        pltpu.VMEM((PPW, NLANES), jnp.float32),
        pltpu.VMEM((IDXP, D), jnp.float32),
        pltpu.VMEM((IDXP, D), jnp.float32),
        pltpu.VMEM((HH, W), jnp.float32),
        pltpu.VMEM((HH, W), jnp.float32),
        pltpu.SemaphoreType.DMA,
        pltpu.SemaphoreType.DMA,
        pltpu.SemaphoreType.DMA,
        pltpu.SemaphoreType.DMA,
    ],
)
def _sc_gather_add(table_hbm, idx_hbm, scal_hbm, out_hbm, idx_v, scal_v,
                   gb0, gb1, am0, am1, gsem0, gsem1, ssem0, ssem1):
    _sc_body(table_hbm, idx_hbm, scal_hbm, out_hbm, idx_v, scal_v,
             gb0, gb1, am0, am1, gsem0, gsem1, ssem0, ssem1)


def kernel(time_step, class_label, cond_3d, time_table, W_proj, b_proj,
           class_table):
    cond2d = _cond2d(time_step, class_label, time_table, W_proj, b_proj,
                     class_table)
    scal = jnp.broadcast_to(
        cond2d.reshape(B * D, 1), (B * D, NLANES))
    flat2 = cond_3d.reshape(2 * B * D, HCHUNK)
    idx = jnp.concatenate(
        [flat2, flat2[:, :IDXP - HCHUNK]], axis=1)
    return _sc_gather_add(class_table, idx, scal)
